# baseline (speedup 1.0000x reference)
import functools
import math

import jax
import jax.numpy as jnp
from jax.experimental import pallas as pl
from jax.experimental.pallas import tpu as pltpu
from jax.experimental.pallas import tpu_sc as plsc

NC = 2
NS = 16
NW = NC * NS
LANES = 16
CHUNK = 128
NBUF = 2


@functools.lru_cache(maxsize=None)
def _build(B, V, D, scale):
    rows_per_w = B // NW
    n_chunks = rows_per_w // CHUNK
    assert n_chunks % NBUF == 0 and n_chunks > 2 * NBUF
    mesh = plsc.VectorSubcoreMesh(
        core_axis_name="c", subcore_axis_name="s",
        num_cores=NC, num_subcores=NS)

    @functools.partial(
        pl.kernel,
        out_type=jax.ShapeDtypeStruct((B, D), jnp.float32),
        mesh=mesh,
        scratch_types=[
            pltpu.VMEM((n_chunks, CHUNK), jnp.int32),
            pltpu.VMEM((NBUF, CHUNK, D), jnp.float32),
            pltpu.VMEM((NBUF, CHUNK, D), jnp.float32),
            pltpu.SemaphoreType.DMA((NBUF,)),
            pltpu.SemaphoreType.DMA((NBUF,)),
        ],
        compiler_params=pltpu.CompilerParams(use_tc_tiling_on_sc=False),
    )
    def emb_kernel(idx_hbm, table_hbm, out_hbm, idx_v, gbuf, sbuf, gsem, ssem):
        wid = jax.lax.axis_index("s") * NC + jax.lax.axis_index("c")
        chunk_base = wid * n_chunks
        row_base = wid * rows_per_w
        pltpu.sync_copy(idx_hbm.at[pl.ds(chunk_base, n_chunks)], idx_v)

        def start_gather(j, b):
            pltpu.make_async_copy(
                table_hbm.at[idx_v.at[j]], gbuf.at[b], gsem.at[b]).start()

        def wait_gather(b):
            pltpu.make_async_copy(
                table_hbm.at[idx_v.at[0]], gbuf.at[b], gsem.at[b]).wait()

        def start_scatter(j, b):
            pltpu.make_async_copy(
                sbuf.at[b], out_hbm.at[pl.ds(row_base + j * CHUNK, CHUNK)],
                ssem.at[b]).start()

        def wait_scatter(b):
            pltpu.make_async_copy(
                sbuf.at[b], out_hbm.at[pl.ds(row_base, CHUNK)],
                ssem.at[b]).wait()

        def scale_chunk(b):
            @pl.loop(0, CHUNK, unroll=8)
            def _row(r):
                for c in range(D // LANES):
                    sl = pl.ds(c * LANES, LANES)
                    sbuf[b, r, sl] = gbuf[b, r, sl] * scale

        for b in range(NBUF):
            start_gather(b, b)
        for j in range(NBUF):
            b = j % NBUF
            wait_gather(b)
            scale_chunk(b)
            start_scatter(j, b)
            start_gather(j + NBUF, b)

        @pl.loop(NBUF, n_chunks, step=NBUF)
        def _group(g):
            for b in range(NBUF):
                j = g + b
                wait_gather(b)
                wait_scatter(b)
                scale_chunk(b)
                start_scatter(j, b)

                @pl.when(j + NBUF < n_chunks)
                def _():
                    start_gather(j + NBUF, b)

        for b in range(NBUF):
            wait_scatter(b)

    return emb_kernel


def kernel(x, table):
    V, D = table.shape
    B = x.size
    scale = math.sqrt(D)
    idx = x.reshape(B // CHUNK, CHUNK).astype(jnp.int32)
    out = _build(B, V, D, scale)(idx, table)
    return out.reshape(x.shape + (D,))
